# Initial kernel scaffold; baseline (speedup 1.0000x reference)
import jax
import jax.numpy as jnp
from jax.experimental import pallas as pl


def kernel(veh_z, ctx, route_z, W1, b1, ln_g, ln_b, W2, b2, gate_W, gate_b, eW1, eb1, eW2, eb2):
    raise NotImplementedError("write your pallas kernel here")


# speedup vs baseline: 1.1420x; 1.1420x over previous
import functools

import jax
import jax.numpy as jnp
from jax.experimental import pallas as pl

NV = 16384
D_HID = 128
D_ROUTE = 64
D_FUSE_HID = 256
D_FUSE_OUT = 192
N_EXPERTS = 8
BLK = 512


def _body(veh_ref, ctx_ref, route_ref, w1a_ref, w1b_ref, w1c_ref, b1_ref,
          ln_g_ref, ln_b_ref, w2_ref, b2_ref, gate_w_ref, gate_b_ref,
          ew1_ref, eb1_ref, ew2_ref, eb2_ref, seg_ref, out_ref):
    z1 = (jnp.dot(veh_ref[...], w1a_ref[...], preferred_element_type=jnp.float32)
          + jnp.dot(ctx_ref[...], w1b_ref[...], preferred_element_type=jnp.float32)
          + jnp.dot(route_ref[...], w1c_ref[...], preferred_element_type=jnp.float32)
          + b1_ref[...])
    h = jax.nn.gelu(z1)
    mu = jnp.mean(h, axis=-1, keepdims=True)
    var = jnp.mean((h - mu) ** 2, axis=-1, keepdims=True)
    hn = (h - mu) / jnp.sqrt(var + 1e-5) * ln_g_ref[...] + ln_b_ref[...]
    f = jnp.dot(hn, w2_ref[...], preferred_element_type=jnp.float32) + b2_ref[...]

    logits = jnp.dot(f, gate_w_ref[...], preferred_element_type=jnp.float32) + gate_b_ref[...]
    lane = jax.lax.broadcasted_iota(jnp.int32, logits.shape, 1)
    v1 = jnp.max(logits, axis=-1, keepdims=True)
    i1 = jnp.min(jnp.where(logits == v1, lane, N_EXPERTS), axis=-1, keepdims=True)
    masked = jnp.where(lane == i1, -jnp.inf, logits)
    v2 = jnp.max(masked, axis=-1, keepdims=True)
    i2 = jnp.min(jnp.where(masked == v2, lane, N_EXPERTS), axis=-1, keepdims=True)
    e2 = jnp.exp(v2 - v1)
    g1 = 1.0 / (1.0 + e2)
    g2 = e2 / (1.0 + e2)

    eh = jax.nn.gelu(jnp.dot(f, ew1_ref[...], preferred_element_type=jnp.float32)
                     + eb1_ref[...])
    ey = (jnp.dot(eh * ew2_ref[...], seg_ref[...],
                  preferred_element_type=jnp.float32) + eb2_ref[...])

    w = jnp.where(lane == i1, g1, 0.0) + jnp.where(lane == i2, g2, 0.0)
    out_ref[...] = jnp.sum(w * ey, axis=-1, keepdims=True)


@functools.partial(jax.jit, static_argnames=("interpret",))
def _run(veh_z, ctx, route_z, W1, b1, ln_g, ln_b, W2, b2, gate_W, gate_b,
         eW1, eb1, eW2, eb2, interpret=False):
    w1a, w1b, w1c = W1[:D_HID], W1[D_HID:2 * D_HID], W1[2 * D_HID:]
    ew1 = eW1.transpose(1, 0, 2).reshape(D_FUSE_OUT, N_EXPERTS * D_FUSE_OUT)
    ew2 = eW2.reshape(1, N_EXPERTS * D_FUSE_OUT)
    eb1f = eb1.reshape(1, N_EXPERTS * D_FUSE_OUT)
    eb2f = eb2.reshape(1, N_EXPERTS)
    seg = jnp.repeat(jnp.eye(N_EXPERTS, dtype=jnp.float32),
                     D_FUSE_OUT, axis=0)

    row = lambda i: (i, 0)
    fixed = lambda i: (0, 0)
    grid = NV // BLK
    out = pl.pallas_call(
        _body,
        grid=(grid,),
        in_specs=[
            pl.BlockSpec((BLK, D_HID), row),
            pl.BlockSpec((BLK, D_HID), row),
            pl.BlockSpec((BLK, D_ROUTE), row),
            pl.BlockSpec((D_HID, D_FUSE_HID), fixed),
            pl.BlockSpec((D_HID, D_FUSE_HID), fixed),
            pl.BlockSpec((D_ROUTE, D_FUSE_HID), fixed),
            pl.BlockSpec((1, D_FUSE_HID), fixed),
            pl.BlockSpec((1, D_FUSE_HID), fixed),
            pl.BlockSpec((1, D_FUSE_HID), fixed),
            pl.BlockSpec((D_FUSE_HID, D_FUSE_OUT), fixed),
            pl.BlockSpec((1, D_FUSE_OUT), fixed),
            pl.BlockSpec((D_FUSE_OUT, N_EXPERTS), fixed),
            pl.BlockSpec((1, N_EXPERTS), fixed),
            pl.BlockSpec((D_FUSE_OUT, N_EXPERTS * D_FUSE_OUT), fixed),
            pl.BlockSpec((1, N_EXPERTS * D_FUSE_OUT), fixed),
            pl.BlockSpec((1, N_EXPERTS * D_FUSE_OUT), fixed),
            pl.BlockSpec((1, N_EXPERTS), fixed),
            pl.BlockSpec((N_EXPERTS * D_FUSE_OUT, N_EXPERTS), fixed),
        ],
        out_specs=pl.BlockSpec((BLK, 1), row),
        out_shape=jax.ShapeDtypeStruct((NV, 1), jnp.float32),
        interpret=interpret,
    )(veh_z, ctx, route_z, w1a, w1b, w1c, b1.reshape(1, -1),
      ln_g.reshape(1, -1), ln_b.reshape(1, -1), W2, b2.reshape(1, -1),
      gate_W, gate_b.reshape(1, -1), ew1, eb1f, ew2, eb2f, seg)
    return out.reshape(NV)


def kernel(veh_z, ctx, route_z, W1, b1, ln_g, ln_b, W2, b2, gate_W, gate_b,
           eW1, eb1, eW2, eb2):
    return _run(veh_z, ctx, route_z, W1, b1, ln_g, ln_b, W2, b2, gate_W,
                gate_b, eW1, eb1, eW2, eb2)


# speedup vs baseline: 1.1653x; 1.0204x over previous
import functools

import jax
import jax.numpy as jnp
from jax.experimental import pallas as pl

NV = 16384
D_HID = 128
D_ROUTE = 64
D_FUSE_HID = 256
D_FUSE_OUT = 192
N_EXPERTS = 8
BLK = 512


def _body(veh_ref, ctx_ref, route_ref, w1a_ref, w1b_ref, w1c_ref, b1_ref,
          ln_g_ref, ln_b_ref, w2_ref, b2_ref, gate_w_ref, gate_b_ref,
          ew1_ref, eb1_ref, ew2_ref, eb2_ref, seg_ref, out_ref):
    z1 = (jnp.dot(veh_ref[...], w1a_ref[...], preferred_element_type=jnp.float32)
          + jnp.dot(ctx_ref[...], w1b_ref[...], preferred_element_type=jnp.float32)
          + jnp.dot(route_ref[...], w1c_ref[...], preferred_element_type=jnp.float32)
          + b1_ref[...])
    h = jax.nn.gelu(z1)
    mu = jnp.mean(h, axis=-1, keepdims=True)
    var = jnp.mean((h - mu) ** 2, axis=-1, keepdims=True)
    hn = (h - mu) / jnp.sqrt(var + 1e-5) * ln_g_ref[...] + ln_b_ref[...]
    f = jnp.dot(hn, w2_ref[...], preferred_element_type=jnp.float32) + b2_ref[...]

    logits = jnp.dot(f, gate_w_ref[...], preferred_element_type=jnp.float32) + gate_b_ref[...]
    lane = jax.lax.broadcasted_iota(jnp.int32, logits.shape, 1)
    v1 = jnp.max(logits, axis=-1, keepdims=True)
    i1 = jnp.min(jnp.where(logits == v1, lane, N_EXPERTS), axis=-1, keepdims=True)
    masked = jnp.where(lane == i1, -jnp.inf, logits)
    v2 = jnp.max(masked, axis=-1, keepdims=True)
    i2 = jnp.min(jnp.where(masked == v2, lane, N_EXPERTS), axis=-1, keepdims=True)
    e2 = jnp.exp(v2 - v1)
    g1 = 1.0 / (1.0 + e2)
    g2 = e2 / (1.0 + e2)

    eh = jax.nn.gelu(jnp.dot(f.astype(jnp.bfloat16), ew1_ref[...],
                             preferred_element_type=jnp.float32)
                     + eb1_ref[...])
    ey = (jnp.dot(eh * ew2_ref[...], seg_ref[...],
                  preferred_element_type=jnp.float32) + eb2_ref[...])

    w = jnp.where(lane == i1, g1, 0.0) + jnp.where(lane == i2, g2, 0.0)
    out_ref[...] = jnp.sum(w * ey, axis=-1, keepdims=True)


@functools.partial(jax.jit, static_argnames=("interpret",))
def _run(veh_z, ctx, route_z, W1, b1, ln_g, ln_b, W2, b2, gate_W, gate_b,
         eW1, eb1, eW2, eb2, interpret=False):
    w1a, w1b, w1c = W1[:D_HID], W1[D_HID:2 * D_HID], W1[2 * D_HID:]
    ew1 = eW1.transpose(1, 0, 2).reshape(
        D_FUSE_OUT, N_EXPERTS * D_FUSE_OUT).astype(jnp.bfloat16)
    ew2 = eW2.reshape(1, N_EXPERTS * D_FUSE_OUT)
    eb1f = eb1.reshape(1, N_EXPERTS * D_FUSE_OUT)
    eb2f = eb2.reshape(1, N_EXPERTS)
    seg = jnp.repeat(jnp.eye(N_EXPERTS, dtype=jnp.float32),
                     D_FUSE_OUT, axis=0)

    row = lambda i: (i, 0)
    fixed = lambda i: (0, 0)
    grid = NV // BLK
    out = pl.pallas_call(
        _body,
        grid=(grid,),
        in_specs=[
            pl.BlockSpec((BLK, D_HID), row),
            pl.BlockSpec((BLK, D_HID), row),
            pl.BlockSpec((BLK, D_ROUTE), row),
            pl.BlockSpec((D_HID, D_FUSE_HID), fixed),
            pl.BlockSpec((D_HID, D_FUSE_HID), fixed),
            pl.BlockSpec((D_ROUTE, D_FUSE_HID), fixed),
            pl.BlockSpec((1, D_FUSE_HID), fixed),
            pl.BlockSpec((1, D_FUSE_HID), fixed),
            pl.BlockSpec((1, D_FUSE_HID), fixed),
            pl.BlockSpec((D_FUSE_HID, D_FUSE_OUT), fixed),
            pl.BlockSpec((1, D_FUSE_OUT), fixed),
            pl.BlockSpec((D_FUSE_OUT, N_EXPERTS), fixed),
            pl.BlockSpec((1, N_EXPERTS), fixed),
            pl.BlockSpec((D_FUSE_OUT, N_EXPERTS * D_FUSE_OUT), fixed),
            pl.BlockSpec((1, N_EXPERTS * D_FUSE_OUT), fixed),
            pl.BlockSpec((1, N_EXPERTS * D_FUSE_OUT), fixed),
            pl.BlockSpec((1, N_EXPERTS), fixed),
            pl.BlockSpec((N_EXPERTS * D_FUSE_OUT, N_EXPERTS), fixed),
        ],
        out_specs=pl.BlockSpec((BLK, 1), row),
        out_shape=jax.ShapeDtypeStruct((NV, 1), jnp.float32),
        interpret=interpret,
    )(veh_z, ctx, route_z, w1a, w1b, w1c, b1.reshape(1, -1),
      ln_g.reshape(1, -1), ln_b.reshape(1, -1), W2, b2.reshape(1, -1),
      gate_W, gate_b.reshape(1, -1), ew1, eb1f, ew2, eb2f, seg)
    return out.reshape(NV)


def kernel(veh_z, ctx, route_z, W1, b1, ln_g, ln_b, W2, b2, gate_W, gate_b,
           eW1, eb1, eW2, eb2):
    return _run(veh_z, ctx, route_z, W1, b1, ln_g, ln_b, W2, b2, gate_W,
                gate_b, eW1, eb1, eW2, eb2)


# speedup vs baseline: 1.2981x; 1.1140x over previous
import functools

import jax
import jax.numpy as jnp
from jax.experimental import pallas as pl

NV = 16384
D_HID = 128
D_ROUTE = 64
D_FUSE_HID = 256
D_FUSE_OUT = 192
N_EXPERTS = 8
BLK = 512


def _body(veh_ref, ctx_ref, route_ref, w1a_ref, w1b_ref, w1c_ref, b1_ref,
          ln_g_ref, ln_b_ref, w2_ref, b2_ref, gate_w_ref, gate_b_ref,
          ew1_ref, eb1_ref, ew2_ref, eb2_ref, seg_ref, out_ref):
    z1 = (jnp.dot(veh_ref[...], w1a_ref[...], preferred_element_type=jnp.float32)
          + jnp.dot(ctx_ref[...], w1b_ref[...], preferred_element_type=jnp.float32)
          + jnp.dot(route_ref[...], w1c_ref[...], preferred_element_type=jnp.float32)
          + b1_ref[...])
    h = jax.nn.gelu(z1)
    mu = jnp.mean(h, axis=-1, keepdims=True)
    var = jnp.mean((h - mu) ** 2, axis=-1, keepdims=True)
    hn = (h - mu) / jnp.sqrt(var + 1e-5) * ln_g_ref[...] + ln_b_ref[...]
    f = jnp.dot(hn, w2_ref[...], preferred_element_type=jnp.float32) + b2_ref[...]

    logits = jnp.dot(f, gate_w_ref[...], preferred_element_type=jnp.float32) + gate_b_ref[...]
    lane = jax.lax.broadcasted_iota(jnp.int32, logits.shape, 1)
    v1 = jnp.max(logits, axis=-1, keepdims=True)
    i1 = jnp.min(jnp.where(logits == v1, lane, N_EXPERTS), axis=-1, keepdims=True)
    masked = jnp.where(lane == i1, -jnp.inf, logits)
    v2 = jnp.max(masked, axis=-1, keepdims=True)
    i2 = jnp.min(jnp.where(masked == v2, lane, N_EXPERTS), axis=-1, keepdims=True)
    e2 = jnp.exp(v2 - v1)
    g1 = 1.0 / (1.0 + e2)
    g2 = e2 / (1.0 + e2)

    pre = (jnp.dot(f.astype(jnp.bfloat16), ew1_ref[...],
                   preferred_element_type=jnp.float32)
           + eb1_ref[...]).astype(jnp.bfloat16)
    eh = jax.nn.gelu(pre)
    ey = (jnp.dot(eh * ew2_ref[...], seg_ref[...],
                  preferred_element_type=jnp.float32) + eb2_ref[...])

    w = jnp.where(lane == i1, g1, 0.0) + jnp.where(lane == i2, g2, 0.0)
    out_ref[...] = jnp.sum(w * ey, axis=-1, keepdims=True)


@functools.partial(jax.jit, static_argnames=("interpret",))
def _run(veh_z, ctx, route_z, W1, b1, ln_g, ln_b, W2, b2, gate_W, gate_b,
         eW1, eb1, eW2, eb2, interpret=False):
    w1a, w1b, w1c = W1[:D_HID], W1[D_HID:2 * D_HID], W1[2 * D_HID:]
    ew1 = eW1.transpose(1, 0, 2).reshape(
        D_FUSE_OUT, N_EXPERTS * D_FUSE_OUT).astype(jnp.bfloat16)
    ew2 = eW2.reshape(1, N_EXPERTS * D_FUSE_OUT).astype(jnp.bfloat16)
    eb1f = eb1.reshape(1, N_EXPERTS * D_FUSE_OUT)
    eb2f = eb2.reshape(1, N_EXPERTS)
    seg = jnp.repeat(jnp.eye(N_EXPERTS, dtype=jnp.bfloat16),
                     D_FUSE_OUT, axis=0)

    row = lambda i: (i, 0)
    fixed = lambda i: (0, 0)
    grid = NV // BLK
    out = pl.pallas_call(
        _body,
        grid=(grid,),
        in_specs=[
            pl.BlockSpec((BLK, D_HID), row),
            pl.BlockSpec((BLK, D_HID), row),
            pl.BlockSpec((BLK, D_ROUTE), row),
            pl.BlockSpec((D_HID, D_FUSE_HID), fixed),
            pl.BlockSpec((D_HID, D_FUSE_HID), fixed),
            pl.BlockSpec((D_ROUTE, D_FUSE_HID), fixed),
            pl.BlockSpec((1, D_FUSE_HID), fixed),
            pl.BlockSpec((1, D_FUSE_HID), fixed),
            pl.BlockSpec((1, D_FUSE_HID), fixed),
            pl.BlockSpec((D_FUSE_HID, D_FUSE_OUT), fixed),
            pl.BlockSpec((1, D_FUSE_OUT), fixed),
            pl.BlockSpec((D_FUSE_OUT, N_EXPERTS), fixed),
            pl.BlockSpec((1, N_EXPERTS), fixed),
            pl.BlockSpec((D_FUSE_OUT, N_EXPERTS * D_FUSE_OUT), fixed),
            pl.BlockSpec((1, N_EXPERTS * D_FUSE_OUT), fixed),
            pl.BlockSpec((1, N_EXPERTS * D_FUSE_OUT), fixed),
            pl.BlockSpec((1, N_EXPERTS), fixed),
            pl.BlockSpec((N_EXPERTS * D_FUSE_OUT, N_EXPERTS), fixed),
        ],
        out_specs=pl.BlockSpec((BLK, 1), row),
        out_shape=jax.ShapeDtypeStruct((NV, 1), jnp.float32),
        interpret=interpret,
    )(veh_z, ctx, route_z, w1a, w1b, w1c, b1.reshape(1, -1),
      ln_g.reshape(1, -1), ln_b.reshape(1, -1), W2, b2.reshape(1, -1),
      gate_W, gate_b.reshape(1, -1), ew1, eb1f, ew2, eb2f, seg)
    return out.reshape(NV)


def kernel(veh_z, ctx, route_z, W1, b1, ln_g, ln_b, W2, b2, gate_W, gate_b,
           eW1, eb1, eW2, eb2):
    return _run(veh_z, ctx, route_z, W1, b1, ln_g, ln_b, W2, b2, gate_W,
                gate_b, eW1, eb1, eW2, eb2)


# speedup vs baseline: 1.4230x; 1.0962x over previous
import functools

import jax
import jax.numpy as jnp
from jax.experimental import pallas as pl

NV = 16384
D_HID = 128
D_ROUTE = 64
D_FUSE_HID = 256
D_FUSE_OUT = 192
N_EXPERTS = 8
BLK = 1024


def _body(veh_ref, ctx_ref, route_ref, w1a_ref, w1b_ref, w1c_ref, b1_ref,
          ln_g_ref, ln_b_ref, w2_ref, b2_ref, gate_w_ref, gate_b_ref,
          ew1_ref, eb1_ref, ew2_ref, eb2_ref, seg_ref, out_ref):
    z1 = (jnp.dot(veh_ref[...], w1a_ref[...], preferred_element_type=jnp.float32)
          + jnp.dot(ctx_ref[...], w1b_ref[...], preferred_element_type=jnp.float32)
          + jnp.dot(route_ref[...], w1c_ref[...], preferred_element_type=jnp.float32)
          + b1_ref[...])
    h = jax.nn.gelu(z1)
    mu = jnp.mean(h, axis=-1, keepdims=True)
    var = jnp.mean((h - mu) ** 2, axis=-1, keepdims=True)
    hn = (h - mu) / jnp.sqrt(var + 1e-5) * ln_g_ref[...] + ln_b_ref[...]
    f = jnp.dot(hn, w2_ref[...], preferred_element_type=jnp.float32) + b2_ref[...]

    logits = jnp.dot(f, gate_w_ref[...], preferred_element_type=jnp.float32) + gate_b_ref[...]
    lane = jax.lax.broadcasted_iota(jnp.int32, logits.shape, 1)
    v1 = jnp.max(logits, axis=-1, keepdims=True)
    i1 = jnp.min(jnp.where(logits == v1, lane, N_EXPERTS), axis=-1, keepdims=True)
    masked = jnp.where(lane == i1, -jnp.inf, logits)
    v2 = jnp.max(masked, axis=-1, keepdims=True)
    i2 = jnp.min(jnp.where(masked == v2, lane, N_EXPERTS), axis=-1, keepdims=True)
    e2 = jnp.exp(v2 - v1)
    g1 = 1.0 / (1.0 + e2)
    g2 = e2 / (1.0 + e2)

    pre = (jnp.dot(f.astype(jnp.bfloat16), ew1_ref[...],
                   preferred_element_type=jnp.float32)
           + eb1_ref[...]).astype(jnp.bfloat16)
    eh = jax.nn.gelu(pre)
    ey = (jnp.dot(eh * ew2_ref[...], seg_ref[...],
                  preferred_element_type=jnp.float32) + eb2_ref[...])

    w = jnp.where(lane == i1, g1, 0.0) + jnp.where(lane == i2, g2, 0.0)
    out_ref[...] = jnp.sum(w * ey, axis=-1, keepdims=True)


@functools.partial(jax.jit, static_argnames=("interpret",))
def _run(veh_z, ctx, route_z, W1, b1, ln_g, ln_b, W2, b2, gate_W, gate_b,
         eW1, eb1, eW2, eb2, interpret=False):
    w1a, w1b, w1c = W1[:D_HID], W1[D_HID:2 * D_HID], W1[2 * D_HID:]
    ew1 = eW1.transpose(1, 0, 2).reshape(
        D_FUSE_OUT, N_EXPERTS * D_FUSE_OUT).astype(jnp.bfloat16)
    ew2 = eW2.reshape(1, N_EXPERTS * D_FUSE_OUT).astype(jnp.bfloat16)
    eb1f = eb1.reshape(1, N_EXPERTS * D_FUSE_OUT)
    eb2f = eb2.reshape(1, N_EXPERTS)
    seg = jnp.repeat(jnp.eye(N_EXPERTS, dtype=jnp.bfloat16),
                     D_FUSE_OUT, axis=0)

    row = lambda i: (i, 0)
    fixed = lambda i: (0, 0)
    grid = NV // BLK
    out = pl.pallas_call(
        _body,
        grid=(grid,),
        in_specs=[
            pl.BlockSpec((BLK, D_HID), row),
            pl.BlockSpec((BLK, D_HID), row),
            pl.BlockSpec((BLK, D_ROUTE), row),
            pl.BlockSpec((D_HID, D_FUSE_HID), fixed),
            pl.BlockSpec((D_HID, D_FUSE_HID), fixed),
            pl.BlockSpec((D_ROUTE, D_FUSE_HID), fixed),
            pl.BlockSpec((1, D_FUSE_HID), fixed),
            pl.BlockSpec((1, D_FUSE_HID), fixed),
            pl.BlockSpec((1, D_FUSE_HID), fixed),
            pl.BlockSpec((D_FUSE_HID, D_FUSE_OUT), fixed),
            pl.BlockSpec((1, D_FUSE_OUT), fixed),
            pl.BlockSpec((D_FUSE_OUT, N_EXPERTS), fixed),
            pl.BlockSpec((1, N_EXPERTS), fixed),
            pl.BlockSpec((D_FUSE_OUT, N_EXPERTS * D_FUSE_OUT), fixed),
            pl.BlockSpec((1, N_EXPERTS * D_FUSE_OUT), fixed),
            pl.BlockSpec((1, N_EXPERTS * D_FUSE_OUT), fixed),
            pl.BlockSpec((1, N_EXPERTS), fixed),
            pl.BlockSpec((N_EXPERTS * D_FUSE_OUT, N_EXPERTS), fixed),
        ],
        out_specs=pl.BlockSpec((BLK, 1), row),
        out_shape=jax.ShapeDtypeStruct((NV, 1), jnp.float32),
        interpret=interpret,
    )(veh_z, ctx, route_z, w1a, w1b, w1c, b1.reshape(1, -1),
      ln_g.reshape(1, -1), ln_b.reshape(1, -1), W2, b2.reshape(1, -1),
      gate_W, gate_b.reshape(1, -1), ew1, eb1f, ew2, eb2f, seg)
    return out.reshape(NV)


def kernel(veh_z, ctx, route_z, W1, b1, ln_g, ln_b, W2, b2, gate_W, gate_b,
           eW1, eb1, eW2, eb2):
    return _run(veh_z, ctx, route_z, W1, b1, ln_g, ln_b, W2, b2, gate_W,
                gate_b, eW1, eb1, eW2, eb2)


# speedup vs baseline: 1.4714x; 1.0340x over previous
import functools

import jax
import jax.numpy as jnp
from jax.experimental import pallas as pl

NV = 16384
D_HID = 128
D_ROUTE = 64
D_FUSE_HID = 256
D_FUSE_OUT = 192
N_EXPERTS = 8
BLK = 2048


def _body(veh_ref, ctx_ref, route_ref, w1a_ref, w1b_ref, w1c_ref, b1_ref,
          ln_g_ref, ln_b_ref, w2_ref, b2_ref, gate_w_ref, gate_b_ref,
          ew1_ref, eb1_ref, ew2_ref, eb2_ref, seg_ref, out_ref):
    z1 = (jnp.dot(veh_ref[...], w1a_ref[...], preferred_element_type=jnp.float32)
          + jnp.dot(ctx_ref[...], w1b_ref[...], preferred_element_type=jnp.float32)
          + jnp.dot(route_ref[...], w1c_ref[...], preferred_element_type=jnp.float32)
          + b1_ref[...])
    h = jax.nn.gelu(z1)
    mu = jnp.mean(h, axis=-1, keepdims=True)
    var = jnp.mean((h - mu) ** 2, axis=-1, keepdims=True)
    hn = (h - mu) / jnp.sqrt(var + 1e-5) * ln_g_ref[...] + ln_b_ref[...]
    f = jnp.dot(hn, w2_ref[...], preferred_element_type=jnp.float32) + b2_ref[...]

    logits = jnp.dot(f, gate_w_ref[...], preferred_element_type=jnp.float32) + gate_b_ref[...]
    lane = jax.lax.broadcasted_iota(jnp.int32, logits.shape, 1)
    v1 = jnp.max(logits, axis=-1, keepdims=True)
    i1 = jnp.min(jnp.where(logits == v1, lane, N_EXPERTS), axis=-1, keepdims=True)
    masked = jnp.where(lane == i1, -jnp.inf, logits)
    v2 = jnp.max(masked, axis=-1, keepdims=True)
    i2 = jnp.min(jnp.where(masked == v2, lane, N_EXPERTS), axis=-1, keepdims=True)
    e2 = jnp.exp(v2 - v1)
    g1 = 1.0 / (1.0 + e2)
    g2 = e2 / (1.0 + e2)

    pre = (jnp.dot(f.astype(jnp.bfloat16), ew1_ref[...],
                   preferred_element_type=jnp.float32)
           + eb1_ref[...]).astype(jnp.bfloat16)
    eh = jax.nn.gelu(pre)
    ey = (jnp.dot(eh * ew2_ref[...], seg_ref[...],
                  preferred_element_type=jnp.float32) + eb2_ref[...])

    w = jnp.where(lane == i1, g1, 0.0) + jnp.where(lane == i2, g2, 0.0)
    out_ref[...] = jnp.sum(w * ey, axis=-1, keepdims=True)


@functools.partial(jax.jit, static_argnames=("interpret",))
def _run(veh_z, ctx, route_z, W1, b1, ln_g, ln_b, W2, b2, gate_W, gate_b,
         eW1, eb1, eW2, eb2, interpret=False):
    w1a, w1b, w1c = W1[:D_HID], W1[D_HID:2 * D_HID], W1[2 * D_HID:]
    ew1 = eW1.transpose(1, 0, 2).reshape(
        D_FUSE_OUT, N_EXPERTS * D_FUSE_OUT).astype(jnp.bfloat16)
    ew2 = eW2.reshape(1, N_EXPERTS * D_FUSE_OUT).astype(jnp.bfloat16)
    eb1f = eb1.reshape(1, N_EXPERTS * D_FUSE_OUT)
    eb2f = eb2.reshape(1, N_EXPERTS)
    seg = jnp.repeat(jnp.eye(N_EXPERTS, dtype=jnp.bfloat16),
                     D_FUSE_OUT, axis=0)

    row = lambda i: (i, 0)
    fixed = lambda i: (0, 0)
    grid = NV // BLK
    out = pl.pallas_call(
        _body,
        grid=(grid,),
        in_specs=[
            pl.BlockSpec((BLK, D_HID), row),
            pl.BlockSpec((BLK, D_HID), row),
            pl.BlockSpec((BLK, D_ROUTE), row),
            pl.BlockSpec((D_HID, D_FUSE_HID), fixed),
            pl.BlockSpec((D_HID, D_FUSE_HID), fixed),
            pl.BlockSpec((D_ROUTE, D_FUSE_HID), fixed),
            pl.BlockSpec((1, D_FUSE_HID), fixed),
            pl.BlockSpec((1, D_FUSE_HID), fixed),
            pl.BlockSpec((1, D_FUSE_HID), fixed),
            pl.BlockSpec((D_FUSE_HID, D_FUSE_OUT), fixed),
            pl.BlockSpec((1, D_FUSE_OUT), fixed),
            pl.BlockSpec((D_FUSE_OUT, N_EXPERTS), fixed),
            pl.BlockSpec((1, N_EXPERTS), fixed),
            pl.BlockSpec((D_FUSE_OUT, N_EXPERTS * D_FUSE_OUT), fixed),
            pl.BlockSpec((1, N_EXPERTS * D_FUSE_OUT), fixed),
            pl.BlockSpec((1, N_EXPERTS * D_FUSE_OUT), fixed),
            pl.BlockSpec((1, N_EXPERTS), fixed),
            pl.BlockSpec((N_EXPERTS * D_FUSE_OUT, N_EXPERTS), fixed),
        ],
        out_specs=pl.BlockSpec((BLK, 1), row),
        out_shape=jax.ShapeDtypeStruct((NV, 1), jnp.float32),
        interpret=interpret,
    )(veh_z, ctx, route_z, w1a, w1b, w1c, b1.reshape(1, -1),
      ln_g.reshape(1, -1), ln_b.reshape(1, -1), W2, b2.reshape(1, -1),
      gate_W, gate_b.reshape(1, -1), ew1, eb1f, ew2, eb2f, seg)
    return out.reshape(NV)


def kernel(veh_z, ctx, route_z, W1, b1, ln_g, ln_b, W2, b2, gate_W, gate_b,
           eW1, eb1, eW2, eb2):
    return _run(veh_z, ctx, route_z, W1, b1, ln_g, ln_b, W2, b2, gate_W,
                gate_b, eW1, eb1, eW2, eb2)


# speedup vs baseline: 1.4924x; 1.0142x over previous
import functools

import jax
import jax.numpy as jnp
from jax.experimental import pallas as pl

NV = 16384
D_HID = 128
D_ROUTE = 64
D_FUSE_HID = 256
D_FUSE_OUT = 192
N_EXPERTS = 8
BLK = 4096


def _body(veh_ref, ctx_ref, route_ref, w1a_ref, w1b_ref, w1c_ref, b1_ref,
          ln_g_ref, ln_b_ref, w2_ref, b2_ref, gate_w_ref, gate_b_ref,
          ew1_ref, eb1_ref, ew2_ref, eb2_ref, seg_ref, out_ref):
    z1 = (jnp.dot(veh_ref[...], w1a_ref[...], preferred_element_type=jnp.float32)
          + jnp.dot(ctx_ref[...], w1b_ref[...], preferred_element_type=jnp.float32)
          + jnp.dot(route_ref[...], w1c_ref[...], preferred_element_type=jnp.float32)
          + b1_ref[...])
    h = jax.nn.gelu(z1)
    mu = jnp.mean(h, axis=-1, keepdims=True)
    var = jnp.mean((h - mu) ** 2, axis=-1, keepdims=True)
    hn = (h - mu) / jnp.sqrt(var + 1e-5) * ln_g_ref[...] + ln_b_ref[...]
    f = jnp.dot(hn, w2_ref[...], preferred_element_type=jnp.float32) + b2_ref[...]

    logits = jnp.dot(f, gate_w_ref[...], preferred_element_type=jnp.float32) + gate_b_ref[...]
    lane = jax.lax.broadcasted_iota(jnp.int32, logits.shape, 1)
    v1 = jnp.max(logits, axis=-1, keepdims=True)
    i1 = jnp.min(jnp.where(logits == v1, lane, N_EXPERTS), axis=-1, keepdims=True)
    masked = jnp.where(lane == i1, -jnp.inf, logits)
    v2 = jnp.max(masked, axis=-1, keepdims=True)
    i2 = jnp.min(jnp.where(masked == v2, lane, N_EXPERTS), axis=-1, keepdims=True)
    e2 = jnp.exp(v2 - v1)
    g1 = 1.0 / (1.0 + e2)
    g2 = e2 / (1.0 + e2)

    pre = (jnp.dot(f.astype(jnp.bfloat16), ew1_ref[...],
                   preferred_element_type=jnp.float32)
           + eb1_ref[...]).astype(jnp.bfloat16)
    eh = jax.nn.gelu(pre)
    ey = (jnp.dot(eh * ew2_ref[...], seg_ref[...],
                  preferred_element_type=jnp.float32) + eb2_ref[...])

    w = jnp.where(lane == i1, g1, 0.0) + jnp.where(lane == i2, g2, 0.0)
    out_ref[...] = jnp.sum(w * ey, axis=-1, keepdims=True)


@functools.partial(jax.jit, static_argnames=("interpret",))
def _run(veh_z, ctx, route_z, W1, b1, ln_g, ln_b, W2, b2, gate_W, gate_b,
         eW1, eb1, eW2, eb2, interpret=False):
    w1a, w1b, w1c = W1[:D_HID], W1[D_HID:2 * D_HID], W1[2 * D_HID:]
    ew1 = eW1.transpose(1, 0, 2).reshape(
        D_FUSE_OUT, N_EXPERTS * D_FUSE_OUT).astype(jnp.bfloat16)
    ew2 = eW2.reshape(1, N_EXPERTS * D_FUSE_OUT).astype(jnp.bfloat16)
    eb1f = eb1.reshape(1, N_EXPERTS * D_FUSE_OUT)
    eb2f = eb2.reshape(1, N_EXPERTS)
    seg = jnp.repeat(jnp.eye(N_EXPERTS, dtype=jnp.bfloat16),
                     D_FUSE_OUT, axis=0)

    row = lambda i: (i, 0)
    fixed = lambda i: (0, 0)
    grid = NV // BLK
    out = pl.pallas_call(
        _body,
        grid=(grid,),
        in_specs=[
            pl.BlockSpec((BLK, D_HID), row),
            pl.BlockSpec((BLK, D_HID), row),
            pl.BlockSpec((BLK, D_ROUTE), row),
            pl.BlockSpec((D_HID, D_FUSE_HID), fixed),
            pl.BlockSpec((D_HID, D_FUSE_HID), fixed),
            pl.BlockSpec((D_ROUTE, D_FUSE_HID), fixed),
            pl.BlockSpec((1, D_FUSE_HID), fixed),
            pl.BlockSpec((1, D_FUSE_HID), fixed),
            pl.BlockSpec((1, D_FUSE_HID), fixed),
            pl.BlockSpec((D_FUSE_HID, D_FUSE_OUT), fixed),
            pl.BlockSpec((1, D_FUSE_OUT), fixed),
            pl.BlockSpec((D_FUSE_OUT, N_EXPERTS), fixed),
            pl.BlockSpec((1, N_EXPERTS), fixed),
            pl.BlockSpec((D_FUSE_OUT, N_EXPERTS * D_FUSE_OUT), fixed),
            pl.BlockSpec((1, N_EXPERTS * D_FUSE_OUT), fixed),
            pl.BlockSpec((1, N_EXPERTS * D_FUSE_OUT), fixed),
            pl.BlockSpec((1, N_EXPERTS), fixed),
            pl.BlockSpec((N_EXPERTS * D_FUSE_OUT, N_EXPERTS), fixed),
        ],
        out_specs=pl.BlockSpec((BLK, 1), row),
        out_shape=jax.ShapeDtypeStruct((NV, 1), jnp.float32),
        interpret=interpret,
    )(veh_z, ctx, route_z, w1a, w1b, w1c, b1.reshape(1, -1),
      ln_g.reshape(1, -1), ln_b.reshape(1, -1), W2, b2.reshape(1, -1),
      gate_W, gate_b.reshape(1, -1), ew1, eb1f, ew2, eb2f, seg)
    return out.reshape(NV)


def kernel(veh_z, ctx, route_z, W1, b1, ln_g, ln_b, W2, b2, gate_W, gate_b,
           eW1, eb1, eW2, eb2):
    return _run(veh_z, ctx, route_z, W1, b1, ln_g, ln_b, W2, b2, gate_W,
                gate_b, eW1, eb1, eW2, eb2)


# speedup vs baseline: 1.7008x; 1.1396x over previous
import functools

import jax
import jax.numpy as jnp
from jax.experimental import pallas as pl

NV = 16384
D_HID = 128
D_ROUTE = 64
D_FUSE_HID = 256
D_FUSE_OUT = 192
N_EXPERTS = 8
BLK = 4096


def _body(veh_ref, ctx_ref, route_ref, w1a_ref, w1b_ref, w1c_ref, b1_ref,
          ln_g_ref, ln_b_ref, w2_ref, b2_ref, gate_w_ref, gate_b_ref,
          ew1_ref, eb1_ref, ew2_ref, eb2_ref, seg_ref, out_ref):
    z1 = (jnp.dot(veh_ref[...], w1a_ref[...], preferred_element_type=jnp.float32)
          + jnp.dot(ctx_ref[...], w1b_ref[...], preferred_element_type=jnp.float32)
          + jnp.dot(route_ref[...], w1c_ref[...], preferred_element_type=jnp.float32)
          + b1_ref[...])
    h = jax.nn.gelu(z1)
    mu = jnp.mean(h, axis=-1, keepdims=True)
    var = jnp.mean((h - mu) ** 2, axis=-1, keepdims=True)
    hn = (h - mu) / jnp.sqrt(var + 1e-5) * ln_g_ref[...] + ln_b_ref[...]
    f = jnp.dot(hn, w2_ref[...], preferred_element_type=jnp.float32) + b2_ref[...]

    lt = jax.lax.dot_general(
        gate_w_ref[...], f, (((0,), (1,)), ((), ())),
        preferred_element_type=jnp.float32) + gate_b_ref[...]
    rowi = jax.lax.broadcasted_iota(jnp.int32, lt.shape, 0)
    v1 = jnp.max(lt, axis=0, keepdims=True)
    i1 = jnp.min(jnp.where(lt == v1, rowi, N_EXPERTS), axis=0, keepdims=True)
    masked = jnp.where(rowi == i1, -jnp.inf, lt)
    v2 = jnp.max(masked, axis=0, keepdims=True)
    i2 = jnp.min(jnp.where(masked == v2, rowi, N_EXPERTS), axis=0, keepdims=True)
    g1 = 1.0 / (1.0 + jnp.exp(v2 - v1))
    g2 = 1.0 - g1
    wt = jnp.where(rowi == i1, g1, 0.0) + jnp.where(rowi == i2, g2, 0.0)

    pre = (jnp.dot(f.astype(jnp.bfloat16), ew1_ref[...],
                   preferred_element_type=jnp.float32).astype(jnp.bfloat16)
           + eb1_ref[...])
    eh = jax.nn.gelu(pre)
    eyt = jax.lax.dot_general(
        seg_ref[...], eh * ew2_ref[...], (((0,), (1,)), ((), ())),
        preferred_element_type=jnp.float32) + eb2_ref[...]

    out_ref[...] = jnp.sum(wt * eyt, axis=0, keepdims=True)[None]


@functools.partial(jax.jit, static_argnames=("interpret",))
def _run(veh_z, ctx, route_z, W1, b1, ln_g, ln_b, W2, b2, gate_W, gate_b,
         eW1, eb1, eW2, eb2, interpret=False):
    w1a, w1b, w1c = W1[:D_HID], W1[D_HID:2 * D_HID], W1[2 * D_HID:]
    ew1 = eW1.transpose(1, 0, 2).reshape(
        D_FUSE_OUT, N_EXPERTS * D_FUSE_OUT).astype(jnp.bfloat16)
    ew2 = eW2.reshape(1, N_EXPERTS * D_FUSE_OUT).astype(jnp.bfloat16)
    eb1f = eb1.reshape(1, N_EXPERTS * D_FUSE_OUT).astype(jnp.bfloat16)
    eb2f = eb2.reshape(N_EXPERTS, 1)
    seg = jnp.repeat(jnp.eye(N_EXPERTS, dtype=jnp.bfloat16),
                     D_FUSE_OUT, axis=0)

    row = lambda i: (i, 0)
    fixed = lambda i: (0, 0)
    grid = NV // BLK
    out = pl.pallas_call(
        _body,
        grid=(grid,),
        in_specs=[
            pl.BlockSpec((BLK, D_HID), row),
            pl.BlockSpec((BLK, D_HID), row),
            pl.BlockSpec((BLK, D_ROUTE), row),
            pl.BlockSpec((D_HID, D_FUSE_HID), fixed),
            pl.BlockSpec((D_HID, D_FUSE_HID), fixed),
            pl.BlockSpec((D_ROUTE, D_FUSE_HID), fixed),
            pl.BlockSpec((1, D_FUSE_HID), fixed),
            pl.BlockSpec((1, D_FUSE_HID), fixed),
            pl.BlockSpec((1, D_FUSE_HID), fixed),
            pl.BlockSpec((D_FUSE_HID, D_FUSE_OUT), fixed),
            pl.BlockSpec((1, D_FUSE_OUT), fixed),
            pl.BlockSpec((D_FUSE_OUT, N_EXPERTS), fixed),
            pl.BlockSpec((N_EXPERTS, 1), fixed),
            pl.BlockSpec((D_FUSE_OUT, N_EXPERTS * D_FUSE_OUT), fixed),
            pl.BlockSpec((1, N_EXPERTS * D_FUSE_OUT), fixed),
            pl.BlockSpec((1, N_EXPERTS * D_FUSE_OUT), fixed),
            pl.BlockSpec((N_EXPERTS, 1), fixed),
            pl.BlockSpec((N_EXPERTS * D_FUSE_OUT, N_EXPERTS), fixed),
        ],
        out_specs=pl.BlockSpec((1, 1, BLK), lambda i: (i, 0, 0)),
        out_shape=jax.ShapeDtypeStruct((grid, 1, BLK), jnp.float32),
        interpret=interpret,
    )(veh_z, ctx, route_z, w1a, w1b, w1c, b1.reshape(1, -1),
      ln_g.reshape(1, -1), ln_b.reshape(1, -1), W2, b2.reshape(1, -1),
      gate_W, gate_b.reshape(N_EXPERTS, 1), ew1, eb1f, ew2, eb2f, seg)
    return out.reshape(NV)


def kernel(veh_z, ctx, route_z, W1, b1, ln_g, ln_b, W2, b2, gate_W, gate_b,
           eW1, eb1, eW2, eb2):
    return _run(veh_z, ctx, route_z, W1, b1, ln_g, ln_b, W2, b2, gate_W,
                gate_b, eW1, eb1, eW2, eb2)
